# initial kernel scaffold (unmeasured)
import jax
import jax.numpy as jnp
from jax import lax
from jax.experimental import pallas as pl
from jax.experimental.pallas import tpu as pltpu

N_DEV = 8
CAP = 102
LANES = 128


def kernel(x, router_W, route_idx, expert_W):
    del router_W
    n_tok, d = x.shape
    n_exp_loc, _, h = expert_W.shape
    n_exp = N_DEV * n_exp_loc

    x_bf = x.astype(jnp.bfloat16)
    w_bf = expert_W.astype(jnp.bfloat16)

    def body(x_ref, route_ref, w_ref, out_ref,
             w_all, cnt_all, copy_sem, w_send, w_recv, c_send, c_recv):
        my = lax.axis_index("i")
        left = lax.rem(my + N_DEV - 1, N_DEV)
        right = lax.rem(my + 1, N_DEV)

        bar = pltpu.get_barrier_semaphore()
        for nbr in (left, right):
            pl.semaphore_signal(
                bar, inc=1,
                device_id=(nbr,), device_id_type=pl.DeviceIdType.MESH,
            )
        pl.semaphore_wait(bar, 2)

        routes = route_ref[...]
        e_ids = lax.broadcasted_iota(jnp.int32, (n_tok, LANES), 1)
        onehot = (routes == e_ids).astype(jnp.int32)
        counts = jnp.sum(onehot, axis=0, keepdims=True)
        cnt_all[pl.ds(my, 1), :, :] = counts.reshape(1, 1, LANES)

        cp = pltpu.make_async_copy(w_ref, w_all.at[my], copy_sem)
        cp.start()
        cp.wait()

        for hp in range(N_DEV - 1):
            org = lax.rem(my - hp + N_DEV, N_DEV)
            w_rd = pltpu.make_async_remote_copy(
                src_ref=w_all.at[org], dst_ref=w_all.at[org],
                send_sem=w_send.at[hp], recv_sem=w_recv.at[hp],
                device_id=(right,), device_id_type=pl.DeviceIdType.MESH,
            )
            c_rd = pltpu.make_async_remote_copy(
                src_ref=cnt_all.at[org], dst_ref=cnt_all.at[org],
                send_sem=c_send.at[hp], recv_sem=c_recv.at[hp],
                device_id=(right,), device_id_type=pl.DeviceIdType.MESH,
            )
            w_rd.start()
            c_rd.start()
            w_rd.wait()
            c_rd.wait()

        cnt = cnt_all[...].reshape(N_DEV, LANES)
        shard_row = lax.broadcasted_iota(jnp.int32, (N_DEV, LANES), 0)
        prefix = jnp.sum(
            jnp.where(shard_row < my, cnt, 0), axis=0, keepdims=True
        )
        csum = jnp.cumsum(onehot, axis=0)
        kept = onehot * ((prefix + csum) <= CAP).astype(jnp.int32)
        kept_bf = kept.astype(jnp.bfloat16)

        xv = x_ref[...]
        acc = jnp.zeros((n_tok, h), jnp.float32)
        for e in range(n_exp):
            m = kept_bf[:, e:e + 1]
            acc = acc + jnp.dot(
                xv * m, w_all[e // n_exp_loc, e % n_exp_loc],
                preferred_element_type=jnp.float32,
            )
        out_ref[...] = acc

    return pl.pallas_call(
        body,
        out_shape=jax.ShapeDtypeStruct((n_tok, h), jnp.float32),
        in_specs=[pl.BlockSpec(memory_space=pltpu.VMEM)] * 3,
        out_specs=pl.BlockSpec(memory_space=pltpu.VMEM),
        scratch_shapes=[
            pltpu.VMEM((N_DEV, n_exp_loc, d, h), jnp.bfloat16),
            pltpu.VMEM((N_DEV, 1, LANES), jnp.int32),
            pltpu.SemaphoreType.DMA,
            pltpu.SemaphoreType.DMA((N_DEV - 1,)),
            pltpu.SemaphoreType.DMA((N_DEV - 1,)),
            pltpu.SemaphoreType.DMA((N_DEV - 1,)),
            pltpu.SemaphoreType.DMA((N_DEV - 1,)),
        ],
        compiler_params=pltpu.CompilerParams(collective_id=0),
    )(x_bf, route_idx, w_bf)


# baseline (device time: 29901 ns/iter reference)
import jax
import jax.numpy as jnp
from jax import lax
from jax.experimental import pallas as pl
from jax.experimental.pallas import tpu as pltpu

N_DEV = 8
CAP = 102
LANES = 128


def kernel(x, router_W, route_idx, expert_W):
    del router_W
    n_tok, d = x.shape
    n_exp_loc, _, h = expert_W.shape
    n_exp = N_DEV * n_exp_loc

    x_bf = x.astype(jnp.bfloat16)
    w_bf = expert_W.astype(jnp.bfloat16)

    def body(x_ref, route_ref, w_ref, out_ref,
             w_all, cnt_all, copy_sem, w_send, w_recv, c_send, c_recv):
        my = lax.axis_index("i")
        left = lax.rem(my + N_DEV - 1, N_DEV)
        right = lax.rem(my + 1, N_DEV)

        bar = pltpu.get_barrier_semaphore()
        for nbr in (left, right):
            pl.semaphore_signal(
                bar, inc=1,
                device_id=(nbr,), device_id_type=pl.DeviceIdType.MESH,
            )
        pl.semaphore_wait(bar, 2)

        routes = route_ref[...]
        e_ids = lax.broadcasted_iota(jnp.int32, (n_tok, LANES), 1)
        onehot = (routes == e_ids).astype(jnp.int32)
        counts = jnp.sum(onehot, axis=0, keepdims=True)
        cnt_all[pl.ds(my, 1), :, :] = counts.reshape(1, 1, LANES)

        cp = pltpu.make_async_copy(w_ref, w_all.at[my], copy_sem)
        cp.start()
        cp.wait()

        for hp in range(N_DEV - 1):
            org = lax.rem(my - hp + N_DEV, N_DEV)
            w_rd = pltpu.make_async_remote_copy(
                src_ref=w_all.at[org], dst_ref=w_all.at[org],
                send_sem=w_send.at[hp], recv_sem=w_recv.at[hp],
                device_id=(right,), device_id_type=pl.DeviceIdType.MESH,
            )
            c_rd = pltpu.make_async_remote_copy(
                src_ref=cnt_all.at[org], dst_ref=cnt_all.at[org],
                send_sem=c_send.at[hp], recv_sem=c_recv.at[hp],
                device_id=(right,), device_id_type=pl.DeviceIdType.MESH,
            )
            w_rd.start()
            c_rd.start()
            w_rd.wait()
            c_rd.wait()

        cnt = cnt_all[...].reshape(N_DEV, LANES)
        shard_row = lax.broadcasted_iota(jnp.int32, (N_DEV, LANES), 0)
        prefix = jnp.sum(
            jnp.where(shard_row < my, cnt, 0), axis=0, keepdims=True
        )
        row = lax.broadcasted_iota(jnp.int32, (n_tok, n_tok), 0)
        col = lax.broadcasted_iota(jnp.int32, (n_tok, n_tok), 1)
        tri = (col <= row).astype(jnp.bfloat16)
        csum = jnp.dot(
            tri, onehot.astype(jnp.bfloat16),
            preferred_element_type=jnp.float32,
        )
        kept = onehot * (
            (prefix.astype(jnp.float32) + csum) <= CAP
        ).astype(jnp.int32)
        kept_bf = kept.astype(jnp.bfloat16)

        xv = x_ref[...]
        acc = jnp.zeros((n_tok, h), jnp.float32)
        for e in range(n_exp):
            m = kept_bf[:, e:e + 1]
            acc = acc + jnp.dot(
                xv * m, w_all[e // n_exp_loc, e % n_exp_loc],
                preferred_element_type=jnp.float32,
            )
        out_ref[...] = acc

    return pl.pallas_call(
        body,
        out_shape=jax.ShapeDtypeStruct((n_tok, h), jnp.float32),
        in_specs=[pl.BlockSpec(memory_space=pltpu.VMEM)] * 3,
        out_specs=pl.BlockSpec(memory_space=pltpu.VMEM),
        scratch_shapes=[
            pltpu.VMEM((N_DEV, n_exp_loc, d, h), jnp.bfloat16),
            pltpu.VMEM((N_DEV, 1, LANES), jnp.int32),
            pltpu.SemaphoreType.DMA,
            pltpu.SemaphoreType.DMA((N_DEV - 1,)),
            pltpu.SemaphoreType.DMA((N_DEV - 1,)),
            pltpu.SemaphoreType.DMA((N_DEV - 1,)),
            pltpu.SemaphoreType.DMA((N_DEV - 1,)),
        ],
        compiler_params=pltpu.CompilerParams(collective_id=0),
    )(x_bf, route_idx, w_bf)


# device time: 16635 ns/iter; 1.7975x vs baseline; 1.7975x over previous
import jax
import jax.numpy as jnp
from jax import lax
from jax.experimental import pallas as pl
from jax.experimental.pallas import tpu as pltpu

N_DEV = 8
CAP = 102
LANES = 128


def kernel(x, router_W, route_idx, expert_W):
    del router_W
    n_tok, d = x.shape
    n_exp_loc, _, h = expert_W.shape

    x_bf = x.astype(jnp.bfloat16)
    w_bf = expert_W.astype(jnp.bfloat16)

    def body(x_ref, route_ref, w_ref, out_ref,
             w_all, cnt_all, copy_sem, w_send, w_recv, c_send, c_recv):
        my = lax.axis_index("i")

        bar = pltpu.get_barrier_semaphore()
        for k in range(1, N_DEV):
            pl.semaphore_signal(
                bar, inc=1,
                device_id=(lax.rem(my + k, N_DEV),),
                device_id_type=pl.DeviceIdType.MESH,
            )
        pl.semaphore_wait(bar, N_DEV - 1)

        routes = route_ref[...]
        e_ids = lax.broadcasted_iota(jnp.int32, (n_tok, LANES), 1)
        onehot = (routes == e_ids).astype(jnp.int32)
        counts = jnp.sum(onehot, axis=0, keepdims=True)
        cnt_all[pl.ds(my, 1), :, :] = counts.reshape(1, 1, LANES)

        cp = pltpu.make_async_copy(w_ref, w_all.at[my], copy_sem)
        cp.start()

        rdmas = []
        for k in range(1, N_DEV):
            dst = lax.rem(my + k, N_DEV)
            w_rd = pltpu.make_async_remote_copy(
                src_ref=w_ref, dst_ref=w_all.at[my],
                send_sem=w_send.at[k], recv_sem=w_recv.at[my],
                device_id=(dst,), device_id_type=pl.DeviceIdType.MESH,
            )
            c_rd = pltpu.make_async_remote_copy(
                src_ref=cnt_all.at[my], dst_ref=cnt_all.at[my],
                send_sem=c_send.at[k], recv_sem=c_recv.at[my],
                device_id=(dst,), device_id_type=pl.DeviceIdType.MESH,
            )
            c_rd.start()
            w_rd.start()
            rdmas.append((w_rd, c_rd))

        for dd in range(N_DEV):
            @pl.when(dd != my)
            def _():
                pltpu.make_async_remote_copy(
                    src_ref=cnt_all.at[dd], dst_ref=cnt_all.at[dd],
                    send_sem=c_send.at[0], recv_sem=c_recv.at[dd],
                    device_id=(my,), device_id_type=pl.DeviceIdType.MESH,
                ).wait_recv()

        cnt = cnt_all[...].reshape(N_DEV, LANES)
        shard_row = lax.broadcasted_iota(jnp.int32, (N_DEV, LANES), 0)
        prefix = jnp.sum(
            jnp.where(shard_row < my, cnt, 0), axis=0, keepdims=True
        )
        row = lax.broadcasted_iota(jnp.int32, (n_tok, n_tok), 0)
        col = lax.broadcasted_iota(jnp.int32, (n_tok, n_tok), 1)
        tri = (col <= row).astype(jnp.bfloat16)
        csum = jnp.dot(
            tri, onehot.astype(jnp.bfloat16),
            preferred_element_type=jnp.float32,
        )
        kept = onehot * (
            (prefix.astype(jnp.float32) + csum) <= CAP
        ).astype(jnp.int32)
        kept_bf = kept.astype(jnp.bfloat16)
        xv = x_ref[...]

        acc = jnp.zeros((n_tok, h), jnp.float32)
        cp.wait()
        for dd in range(N_DEV):
            @pl.when(dd != my)
            def _():
                pltpu.make_async_remote_copy(
                    src_ref=w_all.at[dd], dst_ref=w_all.at[dd],
                    send_sem=w_send.at[0], recv_sem=w_recv.at[dd],
                    device_id=(my,), device_id_type=pl.DeviceIdType.MESH,
                ).wait_recv()
            for s in range(n_exp_loc):
                e = dd * n_exp_loc + s
                m = kept_bf[:, e:e + 1]
                acc = acc + jnp.dot(
                    xv * m, w_all[dd, s],
                    preferred_element_type=jnp.float32,
                )
        out_ref[...] = acc

        for w_rd, c_rd in rdmas:
            w_rd.wait_send()
            c_rd.wait_send()

    return pl.pallas_call(
        body,
        out_shape=jax.ShapeDtypeStruct((n_tok, h), jnp.float32),
        in_specs=[pl.BlockSpec(memory_space=pltpu.VMEM)] * 3,
        out_specs=pl.BlockSpec(memory_space=pltpu.VMEM),
        scratch_shapes=[
            pltpu.VMEM((N_DEV, n_exp_loc, d, h), jnp.bfloat16),
            pltpu.VMEM((N_DEV, 1, LANES), jnp.int32),
            pltpu.SemaphoreType.DMA,
            pltpu.SemaphoreType.DMA((N_DEV,)),
            pltpu.SemaphoreType.DMA((N_DEV,)),
            pltpu.SemaphoreType.DMA((N_DEV,)),
            pltpu.SemaphoreType.DMA((N_DEV,)),
        ],
        compiler_params=pltpu.CompilerParams(collective_id=0),
    )(x_bf, route_idx, w_bf)
